# Initial kernel scaffold; baseline (speedup 1.0000x reference)
import jax
import jax.numpy as jnp
from jax.experimental import pallas as pl


def kernel(input_feat, x, conv_w, conv_b, bn_gamma, bn_beta, offset_w, offset_b, weight_w, weight_b):
    raise NotImplementedError("write your pallas kernel here")


# speedup vs baseline: 4.4581x; 4.4581x over previous
import functools

import jax
import jax.numpy as jnp
from jax import lax
from jax.experimental import pallas as pl
from jax.experimental.pallas import tpu as pltpu
from jax.experimental.pallas import tpu_sc as plsc

B, C, L, K = 8, 384, 4096, 3
LANES = 16
NW = 32
CHUNK = 128
NCHUNK = B * L // CHUNK
CHUNK_PER_W = NCHUNK // NW
NGRP = CHUNK // LANES


def _dense_body(x_ref, cw_ref, cb_ref, g_ref, be_ref, w8_ref, b8_ref,
                a_ref, b_ref, s_ref):
    xb = x_ref[0]
    w0 = cw_ref[:, 0:1]
    w1 = cw_ref[:, 1:2]
    w2 = cw_ref[:, 2:3]
    z = jnp.zeros((C, 1), dtype=jnp.float32)
    xm1 = jnp.concatenate([z, xb[:, :-1]], axis=1)
    xp1 = jnp.concatenate([xb[:, 1:], z], axis=1)
    y = xm1 * w0 + xb * w1 + xp1 * w2 + cb_ref[...]
    y = y * g_ref[...] + be_ref[...]
    feat = 0.5 * y * (1.0 + lax.erf(y * 0.7071067811865476))
    r8 = jnp.dot(w8_ref[...], feat, preferred_element_type=jnp.float32)
    r8 = r8 + b8_ref[...]
    off = r8[0:K]
    logits = r8[K:2 * K]
    m = jnp.max(logits, axis=0, keepdims=True)
    e = jnp.exp(logits - m)
    w = e / jnp.sum(e, axis=0, keepdims=True)
    offc = jnp.clip(off, -float(C), float(C))
    s = jnp.floor(offc)
    f = offc - s
    a_ref[0] = w * (1.0 - f)
    b_ref[0] = w * f
    s_ref[0] = s.astype(jnp.int32)


def _dense_stage(x, conv_w, conv_b, bn_gamma, bn_beta, w8, b8):
    spec_c1 = pl.BlockSpec((C, 1), lambda i: (0, 0))
    out_spec = pl.BlockSpec((1, K, L), lambda i: (i, 0, 0))
    return pl.pallas_call(
        _dense_body,
        grid=(B,),
        in_specs=[
            pl.BlockSpec((1, C, L), lambda i: (i, 0, 0)),
            pl.BlockSpec((C, K), lambda i: (0, 0)),
            spec_c1, spec_c1, spec_c1,
            pl.BlockSpec((2 * K + 2, C), lambda i: (0, 0)),
            pl.BlockSpec((2 * K + 2, 1), lambda i: (0, 0)),
        ],
        out_specs=[out_spec, out_spec, out_spec],
        out_shape=[
            jax.ShapeDtypeStruct((B, K, L), jnp.float32),
            jax.ShapeDtypeStruct((B, K, L), jnp.float32),
            jax.ShapeDtypeStruct((B, K, L), jnp.int32),
        ],
    )(x.reshape(B, C, L), conv_w, conv_b, bn_gamma, bn_beta, w8, b8)


def _sc_body(xt_hbm, a_hbm, b_hbm, s_hbm, out_hbm, xv, ov, av, bv, sv):
    nc = 2
    wid = lax.axis_index("s") * nc + lax.axis_index("c")
    lane = lax.iota(jnp.int32, LANES)

    def blk_body(i, _):
        gblk = wid * CHUNK_PER_W + i
        bb = gblk // (L // CHUNK)
        l0 = (gblk % (L // CHUNK)) * CHUNK
        start = gblk * (CHUNK * C)
        pltpu.sync_copy(xt_hbm.at[pl.ds(start, CHUNK * C)], xv)
        pltpu.sync_copy(a_hbm.at[bb, :, pl.ds(l0, CHUNK)], av)
        pltpu.sync_copy(b_hbm.at[bb, :, pl.ds(l0, CHUNK)], bv)
        pltpu.sync_copy(s_hbm.at[bb, :, pl.ds(l0, CHUNK)], sv)

        for t in range(NGRP):
            lane_t = lane + (t * LANES)
            tok_base = lane_t * C
            lo = tok_base
            hi = tok_base + (C - 1)
            sk = [sv[k, pl.ds(t * LANES, LANES)] for k in range(K)]
            ak = [av[k, pl.ds(t * LANES, LANES)] for k in range(K)]
            bk = [bv[k, pl.ds(t * LANES, LANES)] for k in range(K)]
            skb = [sk[k] + tok_base for k in range(K)]
            g0 = [plsc.load_gather(
                      xv, [jnp.minimum(jnp.maximum(skb[k], lo), hi)])
                  for k in range(K)]

            def c_body(c, gcur, skb=skb, lo=lo, hi=hi, ak=ak, bk=bk,
                       tok_base=tok_base):
                gnext = []
                out_c = jnp.zeros((LANES,), jnp.float32)
                for k in range(K):
                    idx = jnp.minimum(jnp.maximum(skb[k] + (c + 1), lo), hi)
                    gn = plsc.load_gather(xv, [idx])
                    out_c = out_c + ak[k] * gcur[k] + bk[k] * gn
                    gnext.append(gn)
                plsc.store_scatter(ov, [tok_base + c], out_c)
                return gnext

            lax.fori_loop(0, C, c_body, g0, unroll=4)
        pltpu.sync_copy(ov, out_hbm.at[pl.ds(start, CHUNK * C)])
        return 0

    lax.fori_loop(0, CHUNK_PER_W, blk_body, 0)


def _sc_stage(xt, a, b, s):
    mesh = plsc.VectorSubcoreMesh(core_axis_name="c", subcore_axis_name="s")
    f = pl.kernel(
        _sc_body,
        out_type=jax.ShapeDtypeStruct((B * L * C,), jnp.float32),
        mesh=mesh,
        scratch_types=[
            pltpu.VMEM((CHUNK * C,), jnp.float32),
            pltpu.VMEM((CHUNK * C,), jnp.float32),
            pltpu.VMEM((K, CHUNK), jnp.float32),
            pltpu.VMEM((K, CHUNK), jnp.float32),
            pltpu.VMEM((K, CHUNK), jnp.int32),
        ],
        compiler_params=pltpu.CompilerParams(
            use_tc_tiling_on_sc=False, needs_layout_passes=False),
    )
    return f(xt, a, b, s)


def kernel(input_feat, x, conv_w, conv_b, bn_gamma, bn_beta,
           offset_w, offset_b, weight_w, weight_b):
    del input_feat
    cw = conv_w[:, 0, :].astype(jnp.float32)
    w8 = jnp.concatenate(
        [offset_w, weight_w, jnp.zeros((2, C), jnp.float32)], axis=0)
    b8 = jnp.concatenate(
        [offset_b, weight_b, jnp.zeros((2,), jnp.float32)])[:, None]
    a, b, s = _dense_stage(x, cw, conv_b[:, None], bn_gamma[:, None],
                           bn_beta[:, None], w8, b8)
    xt = jnp.transpose(x, (0, 2, 1)).reshape(B * L * C)
    out_t = _sc_stage(xt, a, b, s)
    return jnp.transpose(out_t.reshape(B, L, C), (0, 2, 1))


# speedup vs baseline: 10.9363x; 2.4531x over previous
import functools

import jax
import jax.numpy as jnp
from jax import lax
from jax.experimental import pallas as pl
from jax.experimental.pallas import tpu as pltpu
from jax.experimental.pallas import tpu_sc as plsc

B, C, L, K = 8, 384, 4096, 3
LANES = 16
NW = 32
CHUNK = 128
NCHUNK = B * L // CHUNK
CHUNK_PER_W = NCHUNK // NW
NGRP = CHUNK // LANES


def _dense_body(x_ref, cw_ref, cb_ref, g_ref, be_ref, w8_ref, b8_ref,
                a_ref, b_ref, s_ref):
    xb = x_ref[0]
    w0 = cw_ref[:, 0:1]
    w1 = cw_ref[:, 1:2]
    w2 = cw_ref[:, 2:3]
    z = jnp.zeros((C, 1), dtype=jnp.float32)
    xm1 = jnp.concatenate([z, xb[:, :-1]], axis=1)
    xp1 = jnp.concatenate([xb[:, 1:], z], axis=1)
    y = xm1 * w0 + xb * w1 + xp1 * w2 + cb_ref[...]
    y = y * g_ref[...] + be_ref[...]
    feat = 0.5 * y * (1.0 + lax.erf(y * 0.7071067811865476))
    r8 = jnp.dot(w8_ref[...], feat, preferred_element_type=jnp.float32)
    r8 = r8 + b8_ref[...]
    off = r8[0:K]
    logits = r8[K:2 * K]
    m = jnp.max(logits, axis=0, keepdims=True)
    e = jnp.exp(logits - m)
    w = e / jnp.sum(e, axis=0, keepdims=True)
    offc = jnp.clip(off, -float(C), float(C))
    s = jnp.floor(offc)
    f = offc - s
    a_ref[0] = w * (1.0 - f)
    b_ref[0] = w * f
    s_ref[0] = s.astype(jnp.int32)


def _dense_stage(x, conv_w, conv_b, bn_gamma, bn_beta, w8, b8):
    spec_c1 = pl.BlockSpec((C, 1), lambda i: (0, 0))
    out_spec = pl.BlockSpec((1, K, L), lambda i: (i, 0, 0))
    return pl.pallas_call(
        _dense_body,
        grid=(B,),
        in_specs=[
            pl.BlockSpec((1, C, L), lambda i: (i, 0, 0)),
            pl.BlockSpec((C, K), lambda i: (0, 0)),
            spec_c1, spec_c1, spec_c1,
            pl.BlockSpec((2 * K + 2, C), lambda i: (0, 0)),
            pl.BlockSpec((2 * K + 2, 1), lambda i: (0, 0)),
        ],
        out_specs=[out_spec, out_spec, out_spec],
        out_shape=[
            jax.ShapeDtypeStruct((B, K, L), jnp.float32),
            jax.ShapeDtypeStruct((B, K, L), jnp.float32),
            jax.ShapeDtypeStruct((B, K, L), jnp.int32),
        ],
    )(x.reshape(B, C, L), conv_w, conv_b, bn_gamma, bn_beta, w8, b8)


PAD = 8
WIN = 4
SMAX = 3


def _sc_body(x_hbm, a_hbm, b_hbm, s_hbm, out_hbm, xv, ov, av, bv, sv):
    nc = 2
    wid = lax.axis_index("s") * nc + lax.axis_index("c")
    zero = jnp.zeros((LANES,), jnp.float32)

    def blk_body(i, _):
        gblk = wid * CHUNK_PER_W + i
        bb = gblk // (L // CHUNK)
        l0 = (gblk % (L // CHUNK)) * CHUNK
        pltpu.sync_copy(x_hbm.at[bb, :, pl.ds(l0, CHUNK)],
                        xv.at[pl.ds(PAD, C), :])
        pltpu.sync_copy(a_hbm.at[bb, :, pl.ds(l0, CHUNK)], av)
        pltpu.sync_copy(b_hbm.at[bb, :, pl.ds(l0, CHUNK)], bv)
        pltpu.sync_copy(s_hbm.at[bb, :, pl.ds(l0, CHUNK)], sv)

        for t in range(NGRP):
            sl = pl.ds(t * LANES, LANES)
            lo_row = xv[PAD, sl]
            hi_row = xv[PAD + C - 1, sl]
            for j in range(1, SMAX + 1):
                xv[PAD - j, sl] = lo_row
                xv[PAD + C - 1 + j, sl] = hi_row

        for t in range(NGRP):
            sl = pl.ds(t * LANES, LANES)
            sk = [sv[k, sl] for k in range(K)]
            ak = [av[k, sl] for k in range(K)]
            bk = [bv[k, sl] for k in range(K)]
            smin = jnp.min(jnp.minimum(jnp.minimum(sk[0], sk[1]), sk[2]))
            smax1 = jnp.max(jnp.maximum(jnp.maximum(sk[0], sk[1]), sk[2])) + 1
            ok = ((smin >= -SMAX) & (smax1 <= SMAX)
                  & (smax1 - smin <= WIN - 1))

            def coeff_for(sigma):
                cf = zero
                for k in range(K):
                    cf = cf + jnp.where(sk[k] == sigma, ak[k], zero)
                    cf = cf + jnp.where(sk[k] + 1 == sigma, bk[k], zero)
                return cf

            @pl.when(ok)
            def _fast(sl=sl, sk=sk, ak=ak, bk=bk, smin=smin):
                cfs = [coeff_for(smin + j) for j in range(WIN)]
                base = smin + PAD

                def c_body(c, _):
                    acc = cfs[0] * xv[base + c, sl]
                    for j in range(1, WIN):
                        acc = acc + cfs[j] * xv[base + c + j, sl]
                    ov[c, sl] = acc
                    return 0

                lax.fori_loop(0, C, c_body, 0, unroll=8)

            @pl.when(jnp.logical_not(ok))
            def _general(sl=sl, sk=sk, ak=ak, bk=bk, smin=smin, smax1=smax1):
                def zero_body(c, _):
                    ov[c, sl] = zero
                    return 0
                lax.fori_loop(0, C, zero_body, 0, unroll=8)

                def sigma_body(sigma, _):
                    cf = coeff_for(sigma)

                    def c_body(c, _, cf=cf, sigma=sigma):
                        r = jnp.minimum(jnp.maximum(c + sigma, 0), C - 1)
                        ov[c, sl] = ov[c, sl] + cf * xv[r + PAD, sl]
                        return 0

                    lax.fori_loop(0, C, c_body, 0, unroll=4)
                    return 0

                lax.fori_loop(smin, smax1 + 1, sigma_body, 0)

        pltpu.sync_copy(ov, out_hbm.at[bb, :, pl.ds(l0, CHUNK)])
        return 0

    lax.fori_loop(0, CHUNK_PER_W, blk_body, 0)


def _sc_stage(x, a, b, s):
    mesh = plsc.VectorSubcoreMesh(core_axis_name="c", subcore_axis_name="s")
    f = pl.kernel(
        _sc_body,
        out_type=jax.ShapeDtypeStruct((B, C, L), jnp.float32),
        mesh=mesh,
        scratch_types=[
            pltpu.VMEM((C + 2 * PAD, CHUNK), jnp.float32),
            pltpu.VMEM((C, CHUNK), jnp.float32),
            pltpu.VMEM((K, CHUNK), jnp.float32),
            pltpu.VMEM((K, CHUNK), jnp.float32),
            pltpu.VMEM((K, CHUNK), jnp.int32),
        ],
        compiler_params=pltpu.CompilerParams(
            use_tc_tiling_on_sc=False, needs_layout_passes=False),
    )
    return f(x, a, b, s)


def kernel(input_feat, x, conv_w, conv_b, bn_gamma, bn_beta,
           offset_w, offset_b, weight_w, weight_b):
    del input_feat
    cw = conv_w[:, 0, :].astype(jnp.float32)
    w8 = jnp.concatenate(
        [offset_w, weight_w, jnp.zeros((2, C), jnp.float32)], axis=0)
    b8 = jnp.concatenate(
        [offset_b, weight_b, jnp.zeros((2,), jnp.float32)])[:, None]
    a, b, s = _dense_stage(x, cw, conv_b[:, None], bn_gamma[:, None],
                           bn_beta[:, None], w8, b8)
    return _sc_stage(x, a, b, s)


# speedup vs baseline: 18.9808x; 1.7356x over previous
import functools

import jax
import jax.numpy as jnp
from jax import lax
from jax.experimental import pallas as pl
from jax.experimental.pallas import tpu as pltpu
from jax.experimental.pallas import tpu_sc as plsc

B, C, L, K = 8, 384, 4096, 3
LANES = 16
NW = 32
CHUNK = 128
NCHUNK = B * L // CHUNK
CHUNK_PER_W = NCHUNK // NW
NGRP = CHUNK // LANES


def _dense_body(x_ref, cw_ref, cb_ref, g_ref, be_ref, w8_ref, b8_ref,
                a_ref, b_ref, s_ref):
    xb = x_ref[0]
    w0 = cw_ref[:, 0:1]
    w1 = cw_ref[:, 1:2]
    w2 = cw_ref[:, 2:3]
    z = jnp.zeros((C, 1), dtype=jnp.float32)
    xm1 = jnp.concatenate([z, xb[:, :-1]], axis=1)
    xp1 = jnp.concatenate([xb[:, 1:], z], axis=1)
    y = xm1 * w0 + xb * w1 + xp1 * w2 + cb_ref[...]
    y = y * g_ref[...] + be_ref[...]
    feat = 0.5 * y * (1.0 + lax.erf(y * 0.7071067811865476))
    r8 = jnp.dot(w8_ref[...], feat, preferred_element_type=jnp.float32)
    r8 = r8 + b8_ref[...]
    off = r8[0:K]
    logits = r8[K:2 * K]
    m = jnp.max(logits, axis=0, keepdims=True)
    e = jnp.exp(logits - m)
    w = e / jnp.sum(e, axis=0, keepdims=True)
    offc = jnp.clip(off, -float(C), float(C))
    s = jnp.floor(offc)
    f = offc - s
    a_ref[0] = w * (1.0 - f)
    b_ref[0] = w * f
    s_ref[0] = s.astype(jnp.int32)


def _dense_stage(x, conv_w, conv_b, bn_gamma, bn_beta, w8, b8):
    spec_c1 = pl.BlockSpec((C, 1), lambda i: (0, 0))
    out_spec = pl.BlockSpec((1, K, L), lambda i: (i, 0, 0))
    return pl.pallas_call(
        _dense_body,
        grid=(B,),
        in_specs=[
            pl.BlockSpec((1, C, L), lambda i: (i, 0, 0)),
            pl.BlockSpec((C, K), lambda i: (0, 0)),
            spec_c1, spec_c1, spec_c1,
            pl.BlockSpec((2 * K + 2, C), lambda i: (0, 0)),
            pl.BlockSpec((2 * K + 2, 1), lambda i: (0, 0)),
        ],
        out_specs=[out_spec, out_spec, out_spec],
        out_shape=[
            jax.ShapeDtypeStruct((B, K, L), jnp.float32),
            jax.ShapeDtypeStruct((B, K, L), jnp.float32),
            jax.ShapeDtypeStruct((B, K, L), jnp.int32),
        ],
    )(x.reshape(B, C, L), conv_w, conv_b, bn_gamma, bn_beta, w8, b8)


PAD = 8
WIN = 4
SMAX = 3


def _sc_body(x_hbm, a_hbm, b_hbm, s_hbm, out_hbm, xv, ov, av, bv, sv):
    nc = 2
    wid = lax.axis_index("s") * nc + lax.axis_index("c")
    zero = jnp.zeros((LANES,), jnp.float32)

    def blk_body(i, _):
        gblk = wid * CHUNK_PER_W + i
        bb = gblk // (L // CHUNK)
        l0 = (gblk % (L // CHUNK)) * CHUNK
        pltpu.sync_copy(x_hbm.at[bb, :, pl.ds(l0, CHUNK)],
                        xv.at[pl.ds(PAD, C), :])
        pltpu.sync_copy(a_hbm.at[bb, :, pl.ds(l0, CHUNK)], av)
        pltpu.sync_copy(b_hbm.at[bb, :, pl.ds(l0, CHUNK)], bv)
        pltpu.sync_copy(s_hbm.at[bb, :, pl.ds(l0, CHUNK)], sv)

        for t in range(NGRP):
            sl = pl.ds(t * LANES, LANES)
            lo_row = xv[PAD, sl]
            hi_row = xv[PAD + C - 1, sl]
            for j in range(1, SMAX + 1):
                xv[PAD - j, sl] = lo_row
                xv[PAD + C - 1 + j, sl] = hi_row

        for t in range(NGRP):
            sl = pl.ds(t * LANES, LANES)
            sk = [sv[k, sl] for k in range(K)]
            ak = [av[k, sl] for k in range(K)]
            bk = [bv[k, sl] for k in range(K)]
            smin = jnp.min(jnp.minimum(jnp.minimum(sk[0], sk[1]), sk[2]))
            smax1 = jnp.max(jnp.maximum(jnp.maximum(sk[0], sk[1]), sk[2])) + 1
            ok = ((smin >= -SMAX) & (smax1 <= SMAX)
                  & (smax1 - smin <= WIN - 1))

            def coeff_for(sigma):
                cf = zero
                for k in range(K):
                    cf = cf + jnp.where(sk[k] == sigma, ak[k], zero)
                    cf = cf + jnp.where(sk[k] + 1 == sigma, bk[k], zero)
                return cf

            @pl.when(ok)
            def _fast(sl=sl, sk=sk, ak=ak, bk=bk, smin=smin):
                cfs = [coeff_for(smin + j) for j in range(WIN)]
                base = smin + PAD
                r0 = xv[base, sl]
                r1 = xv[base + 1, sl]
                r2 = xv[base + 2, sl]

                @plsc.parallel_loop(0, C, unroll=8, carry=(r0, r1, r2))
                def c_body(c, rows):
                    ra, rb, rc = rows
                    rd = xv[base + c + 3, sl]
                    ov[c, sl] = ((cfs[0] * ra + cfs[1] * rb)
                                 + (cfs[2] * rc + cfs[3] * rd))
                    return (rb, rc, rd)

            @pl.when(jnp.logical_not(ok))
            def _general(sl=sl, sk=sk, ak=ak, bk=bk, smin=smin, smax1=smax1):
                @plsc.parallel_loop(0, C, unroll=8)
                def zero_body(c):
                    ov[c, sl] = zero

                def sigma_body(sigma, _):
                    cf = coeff_for(sigma)

                    def c_body(c, _, cf=cf, sigma=sigma):
                        r = jnp.minimum(jnp.maximum(c + sigma, 0), C - 1)
                        ov[c, sl] = ov[c, sl] + cf * xv[r + PAD, sl]
                        return 0

                    lax.fori_loop(0, C, c_body, 0, unroll=4)
                    return 0

                lax.fori_loop(smin, smax1 + 1, sigma_body, 0)

        pltpu.sync_copy(ov, out_hbm.at[bb, :, pl.ds(l0, CHUNK)])
        return 0

    lax.fori_loop(0, CHUNK_PER_W, blk_body, 0)


def _sc_stage(x, a, b, s):
    mesh = plsc.VectorSubcoreMesh(core_axis_name="c", subcore_axis_name="s")
    f = pl.kernel(
        _sc_body,
        out_type=jax.ShapeDtypeStruct((B, C, L), jnp.float32),
        mesh=mesh,
        scratch_types=[
            pltpu.VMEM((C + 2 * PAD, CHUNK), jnp.float32),
            pltpu.VMEM((C, CHUNK), jnp.float32),
            pltpu.VMEM((K, CHUNK), jnp.float32),
            pltpu.VMEM((K, CHUNK), jnp.float32),
            pltpu.VMEM((K, CHUNK), jnp.int32),
        ],
        compiler_params=pltpu.CompilerParams(
            use_tc_tiling_on_sc=False, needs_layout_passes=False),
    )
    return f(x, a, b, s)


def kernel(input_feat, x, conv_w, conv_b, bn_gamma, bn_beta,
           offset_w, offset_b, weight_w, weight_b):
    del input_feat
    cw = conv_w[:, 0, :].astype(jnp.float32)
    w8 = jnp.concatenate(
        [offset_w, weight_w, jnp.zeros((2, C), jnp.float32)], axis=0)
    b8 = jnp.concatenate(
        [offset_b, weight_b, jnp.zeros((2,), jnp.float32)])[:, None]
    a, b, s = _dense_stage(x, cw, conv_b[:, None], bn_gamma[:, None],
                           bn_beta[:, None], w8, b8)
    return _sc_stage(x, a, b, s)


# speedup vs baseline: 25.7062x; 1.3543x over previous
import functools

import jax
import jax.numpy as jnp
from jax import lax
from jax.experimental import pallas as pl
from jax.experimental.pallas import tpu as pltpu
from jax.experimental.pallas import tpu_sc as plsc

B, C, L, K = 8, 384, 4096, 3
LANES = 16
NW = 32
CHUNK = 128
NCHUNK = B * L // CHUNK
CHUNK_PER_W = NCHUNK // NW
NGRP = CHUNK // LANES


def _dense_body(x_ref, cw_ref, cb_ref, g_ref, be_ref, w8_ref, b8_ref,
                a_ref, b_ref, s_ref):
    xb = x_ref[0]
    w0 = cw_ref[:, 0:1]
    w1 = cw_ref[:, 1:2]
    w2 = cw_ref[:, 2:3]
    z = jnp.zeros((C, 1), dtype=jnp.float32)
    xm1 = jnp.concatenate([z, xb[:, :-1]], axis=1)
    xp1 = jnp.concatenate([xb[:, 1:], z], axis=1)
    y = xm1 * w0 + xb * w1 + xp1 * w2 + cb_ref[...]
    y = y * g_ref[...] + be_ref[...]
    feat = 0.5 * y * (1.0 + lax.erf(y * 0.7071067811865476))
    r8 = jnp.dot(w8_ref[...], feat, preferred_element_type=jnp.float32)
    r8 = r8 + b8_ref[...]
    off = r8[0:K]
    logits = r8[K:2 * K]
    m = jnp.max(logits, axis=0, keepdims=True)
    e = jnp.exp(logits - m)
    w = e / jnp.sum(e, axis=0, keepdims=True)
    offc = jnp.clip(off, -float(C), float(C))
    s = jnp.floor(offc)
    f = offc - s
    a_ref[0] = w * (1.0 - f)
    b_ref[0] = w * f
    s_ref[0] = s.astype(jnp.int32)


def _dense_stage(x, conv_w, conv_b, bn_gamma, bn_beta, w8, b8):
    spec_c1 = pl.BlockSpec((C, 1), lambda i: (0, 0))
    out_spec = pl.BlockSpec((1, K, L), lambda i: (i, 0, 0))
    return pl.pallas_call(
        _dense_body,
        grid=(B,),
        in_specs=[
            pl.BlockSpec((1, C, L), lambda i: (i, 0, 0)),
            pl.BlockSpec((C, K), lambda i: (0, 0)),
            spec_c1, spec_c1, spec_c1,
            pl.BlockSpec((2 * K + 2, C), lambda i: (0, 0)),
            pl.BlockSpec((2 * K + 2, 1), lambda i: (0, 0)),
        ],
        out_specs=[out_spec, out_spec, out_spec],
        out_shape=[
            jax.ShapeDtypeStruct((B, K, L), jnp.float32),
            jax.ShapeDtypeStruct((B, K, L), jnp.float32),
            jax.ShapeDtypeStruct((B, K, L), jnp.int32),
        ],
    )(x.reshape(B, C, L), conv_w, conv_b, bn_gamma, bn_beta, w8, b8)


PAD = 8
WIN = 4
SMAX = 3


def _sc_body(x_hbm, a_hbm, b_hbm, s_hbm, out_hbm, xv, ov, av, bv, sv):
    nc = 2
    wid = lax.axis_index("s") * nc + lax.axis_index("c")
    zero = jnp.zeros((LANES,), jnp.float32)

    def blk_body(i, _):
        gblk = wid * CHUNK_PER_W + i
        bb = gblk // (L // CHUNK)
        l0 = (gblk % (L // CHUNK)) * CHUNK
        pltpu.sync_copy(x_hbm.at[bb, :, pl.ds(l0, CHUNK)],
                        xv.at[pl.ds(PAD, C), :])
        pltpu.sync_copy(a_hbm.at[bb, :, pl.ds(l0, CHUNK)], av)
        pltpu.sync_copy(b_hbm.at[bb, :, pl.ds(l0, CHUNK)], bv)
        pltpu.sync_copy(s_hbm.at[bb, :, pl.ds(l0, CHUNK)], sv)

        for t in range(NGRP):
            sl = pl.ds(t * LANES, LANES)
            lo_row = xv[PAD, sl]
            hi_row = xv[PAD + C - 1, sl]
            for j in range(1, SMAX + 1):
                xv[PAD - j, sl] = lo_row
                xv[PAD + C - 1 + j, sl] = hi_row

        for t in range(NGRP):
            sl = pl.ds(t * LANES, LANES)
            sk = [sv[k, sl] for k in range(K)]
            ak = [av[k, sl] for k in range(K)]
            bk = [bv[k, sl] for k in range(K)]
            smin = jnp.min(jnp.minimum(jnp.minimum(sk[0], sk[1]), sk[2]))
            smax1 = jnp.max(jnp.maximum(jnp.maximum(sk[0], sk[1]), sk[2])) + 1
            ok = ((smin >= -SMAX) & (smax1 <= SMAX)
                  & (smax1 - smin <= WIN - 1))

            def coeff_for(sigma):
                cf = zero
                for k in range(K):
                    cf = cf + jnp.where(sk[k] == sigma, ak[k], zero)
                    cf = cf + jnp.where(sk[k] + 1 == sigma, bk[k], zero)
                return cf

            @pl.when(ok)
            def _fast(sl=sl, sk=sk, ak=ak, bk=bk, smin=smin):
                cfs = [coeff_for(smin + j) for j in range(WIN)]
                base = smin + PAD
                r0 = xv[base, sl]
                r1 = xv[base + 1, sl]
                r2 = xv[base + 2, sl]

                @plsc.parallel_loop(0, C, unroll=8, carry=(r0, r1, r2))
                def c_body(c, rows):
                    ra, rb, rc = rows
                    rd = xv[base + c + 3, sl]
                    ov[c, sl] = ((cfs[0] * ra + cfs[1] * rb)
                                 + (cfs[2] * rc + cfs[3] * rd))
                    return (rb, rc, rd)

            @pl.when(jnp.logical_not(ok))
            def _general(sl=sl, sk=sk, ak=ak, bk=bk, smin=smin, smax1=smax1):
                @plsc.parallel_loop(0, C, unroll=8)
                def zero_body(c):
                    ov[c, sl] = zero

                def sigma_body(sigma, _):
                    cf = coeff_for(sigma)

                    def c_body(c, _, cf=cf, sigma=sigma):
                        r = jnp.minimum(jnp.maximum(c + sigma, 0), C - 1)
                        ov[c, sl] = ov[c, sl] + cf * xv[r + PAD, sl]
                        return 0

                    lax.fori_loop(0, C, c_body, 0, unroll=4)
                    return 0

                lax.fori_loop(smin, smax1 + 1, sigma_body, 0)

        pltpu.sync_copy(ov, out_hbm.at[bb, :, pl.ds(l0, CHUNK)])
        return 0

    lax.fori_loop(0, CHUNK_PER_W, blk_body, 0)


def _sc_stage(x, a, b, s):
    mesh = plsc.VectorSubcoreMesh(core_axis_name="c", subcore_axis_name="s")
    f = pl.kernel(
        _sc_body,
        out_type=jax.ShapeDtypeStruct((B, C, L), jnp.float32),
        mesh=mesh,
        scratch_types=[
            pltpu.VMEM((C + 2 * PAD, CHUNK), jnp.float32),
            pltpu.VMEM((C, CHUNK), jnp.float32),
            pltpu.VMEM((K, CHUNK), jnp.float32),
            pltpu.VMEM((K, CHUNK), jnp.float32),
            pltpu.VMEM((K, CHUNK), jnp.int32),
        ],
        compiler_params=pltpu.CompilerParams(needs_layout_passes=False),
    )
    return f(x, a, b, s)


def kernel(input_feat, x, conv_w, conv_b, bn_gamma, bn_beta,
           offset_w, offset_b, weight_w, weight_b):
    del input_feat
    cw = conv_w[:, 0, :].astype(jnp.float32)
    w8 = jnp.concatenate(
        [offset_w, weight_w, jnp.zeros((2, C), jnp.float32)], axis=0)
    b8 = jnp.concatenate(
        [offset_b, weight_b, jnp.zeros((2,), jnp.float32)])[:, None]
    a, b, s = _dense_stage(x, cw, conv_b[:, None], bn_gamma[:, None],
                           bn_beta[:, None], w8, b8)
    return _sc_stage(x, a, b, s)


# speedup vs baseline: 28.5628x; 1.1111x over previous
import functools

import jax
import jax.numpy as jnp
from jax import lax
from jax.experimental import pallas as pl
from jax.experimental.pallas import tpu as pltpu
from jax.experimental.pallas import tpu_sc as plsc

B, C, L, K = 8, 384, 4096, 3
LANES = 16
NW = 32
CHUNK = 128
NCHUNK = B * L // CHUNK
CHUNK_PER_W = NCHUNK // NW
NGRP = CHUNK // LANES


def _dense_body(x_ref, cw_ref, cb_ref, g_ref, be_ref, w8_ref, b8_ref,
                a_ref, b_ref, s_ref):
    xb = x_ref[0]
    w0 = cw_ref[:, 0:1]
    w1 = cw_ref[:, 1:2]
    w2 = cw_ref[:, 2:3]
    z = jnp.zeros((C, 1), dtype=jnp.float32)
    xm1 = jnp.concatenate([z, xb[:, :-1]], axis=1)
    xp1 = jnp.concatenate([xb[:, 1:], z], axis=1)
    y = xm1 * w0 + xb * w1 + xp1 * w2 + cb_ref[...]
    y = y * g_ref[...] + be_ref[...]
    feat = 0.5 * y * (1.0 + lax.erf(y * 0.7071067811865476))
    r8 = jnp.dot(w8_ref[...], feat, preferred_element_type=jnp.float32)
    r8 = r8 + b8_ref[...]
    off = r8[0:K]
    logits = r8[K:2 * K]
    m = jnp.max(logits, axis=0, keepdims=True)
    e = jnp.exp(logits - m)
    w = e / jnp.sum(e, axis=0, keepdims=True)
    offc = jnp.clip(off, -float(C), float(C))
    s = jnp.floor(offc)
    f = offc - s
    a_ref[0] = w * (1.0 - f)
    b_ref[0] = w * f
    s_ref[0] = s.astype(jnp.int32)


def _dense_stage(x, conv_w, conv_b, bn_gamma, bn_beta, w8, b8):
    spec_c1 = pl.BlockSpec((C, 1), lambda i: (0, 0))
    out_spec = pl.BlockSpec((1, K, L), lambda i: (i, 0, 0))
    return pl.pallas_call(
        _dense_body,
        grid=(B,),
        in_specs=[
            pl.BlockSpec((1, C, L), lambda i: (i, 0, 0)),
            pl.BlockSpec((C, K), lambda i: (0, 0)),
            spec_c1, spec_c1, spec_c1,
            pl.BlockSpec((2 * K + 2, C), lambda i: (0, 0)),
            pl.BlockSpec((2 * K + 2, 1), lambda i: (0, 0)),
        ],
        out_specs=[out_spec, out_spec, out_spec],
        out_shape=[
            jax.ShapeDtypeStruct((B, K, L), jnp.float32),
            jax.ShapeDtypeStruct((B, K, L), jnp.float32),
            jax.ShapeDtypeStruct((B, K, L), jnp.int32),
        ],
    )(x.reshape(B, C, L), conv_w, conv_b, bn_gamma, bn_beta, w8, b8)


PAD = 8
WIN = 4
SMAX = 3


HALF = C // 2


def _sc_body(x_hbm, a_hbm, b_hbm, s_hbm, out_hbm, xv, ov, av, bv, sv,
             sem_in, sem_oa, sem_ob):
    nc = 2
    wid = lax.axis_index("s") * nc + lax.axis_index("c")
    zero = jnp.zeros((LANES,), jnp.float32)

    def chunk_coords(i):
        gblk = wid * CHUNK_PER_W + i
        bb = gblk // (L // CHUNK)
        l0 = (gblk % (L // CHUNK)) * CHUNK
        return bb, l0

    def in_copy(i):
        bb, l0 = chunk_coords(i)
        return pltpu.make_async_copy(
            x_hbm.at[bb, :, pl.ds(l0, CHUNK)], xv.at[pl.ds(PAD, C), :],
            sem_in)

    def out_copy(i, h):
        bb, l0 = chunk_coords(i)
        return pltpu.make_async_copy(
            ov.at[pl.ds(h * HALF, HALF), :],
            out_hbm.at[bb, pl.ds(h * HALF, HALF), pl.ds(l0, CHUNK)],
            sem_oa if h == 0 else sem_ob)

    def compute_half(h):
        c0 = h * HALF

        def t_body(t, _):
            sl = pl.ds(t * LANES, LANES)
            sk = [sv[k, sl] for k in range(K)]
            ak = [av[k, sl] for k in range(K)]
            bk = [bv[k, sl] for k in range(K)]
            smin = jnp.min(jnp.minimum(jnp.minimum(sk[0], sk[1]), sk[2]))
            smax1 = jnp.max(jnp.maximum(jnp.maximum(sk[0], sk[1]),
                                        sk[2])) + 1
            ok = ((smin >= -SMAX) & (smax1 <= SMAX)
                  & (smax1 - smin <= WIN - 1))

            def coeff_for(sigma):
                cf = zero
                for k in range(K):
                    cf = cf + jnp.where(sk[k] == sigma, ak[k], zero)
                    cf = cf + jnp.where(sk[k] + 1 == sigma, bk[k], zero)
                return cf

            @pl.when(ok)
            def _fast():
                cfs = [coeff_for(smin + j) for j in range(WIN)]
                base = smin + PAD
                r0 = xv[base + c0, sl]
                r1 = xv[base + c0 + 1, sl]
                r2 = xv[base + c0 + 2, sl]

                @plsc.parallel_loop(c0, c0 + HALF, unroll=8,
                                    carry=(r0, r1, r2))
                def c_body(c, rows):
                    ra, rb, rc = rows
                    rd = xv[base + c + 3, sl]
                    ov[c, sl] = ((cfs[0] * ra + cfs[1] * rb)
                                 + (cfs[2] * rc + cfs[3] * rd))
                    return (rb, rc, rd)

            @pl.when(jnp.logical_not(ok))
            def _general():
                @plsc.parallel_loop(c0, c0 + HALF, unroll=8)
                def zero_body(c):
                    ov[c, sl] = zero

                def sigma_body(sigma, _):
                    cf = coeff_for(sigma)

                    def c_body(c, _, cf=cf, sigma=sigma):
                        r = jnp.minimum(jnp.maximum(c + sigma, 0), C - 1)
                        ov[c, sl] = ov[c, sl] + cf * xv[r + PAD, sl]
                        return 0

                    lax.fori_loop(c0, c0 + HALF, c_body, 0, unroll=4)
                    return 0

                lax.fori_loop(smin, smax1 + 1, sigma_body, 0)

            return 0

        lax.fori_loop(0, NGRP, t_body, 0)

    in_copy(0).start()

    def blk_body(i, _):
        bb, l0 = chunk_coords(i)
        pltpu.sync_copy(a_hbm.at[bb, :, pl.ds(l0, CHUNK)], av)
        pltpu.sync_copy(b_hbm.at[bb, :, pl.ds(l0, CHUNK)], bv)
        pltpu.sync_copy(s_hbm.at[bb, :, pl.ds(l0, CHUNK)], sv)
        in_copy(i).wait()

        def pad_body(t, _):
            sl = pl.ds(t * LANES, LANES)
            lo_row = xv[PAD, sl]
            hi_row = xv[PAD + C - 1, sl]
            for j in range(1, SMAX + 1):
                xv[PAD - j, sl] = lo_row
                xv[PAD + C - 1 + j, sl] = hi_row
            return 0

        lax.fori_loop(0, NGRP, pad_body, 0)

        @pl.when(i > 0)
        def _drain_oa():
            out_copy(i, 0).wait()
        compute_half(0)
        out_copy(i, 0).start()

        @pl.when(i > 0)
        def _drain_ob():
            out_copy(i, 1).wait()
        compute_half(1)
        out_copy(i, 1).start()

        @pl.when(i + 1 < CHUNK_PER_W)
        def _next_in():
            in_copy(i + 1).start()
        return 0

    lax.fori_loop(0, CHUNK_PER_W, blk_body, 0)
    out_copy(CHUNK_PER_W - 1, 0).wait()
    out_copy(CHUNK_PER_W - 1, 1).wait()


def _sc_stage(x, a, b, s):
    mesh = plsc.VectorSubcoreMesh(core_axis_name="c", subcore_axis_name="s")
    f = pl.kernel(
        _sc_body,
        out_type=jax.ShapeDtypeStruct((B, C, L), jnp.float32),
        mesh=mesh,
        scratch_types=[
            pltpu.VMEM((C + 2 * PAD, CHUNK), jnp.float32),
            pltpu.VMEM((C, CHUNK), jnp.float32),
            pltpu.VMEM((K, CHUNK), jnp.float32),
            pltpu.VMEM((K, CHUNK), jnp.float32),
            pltpu.VMEM((K, CHUNK), jnp.int32),
            pltpu.SemaphoreType.DMA,
            pltpu.SemaphoreType.DMA,
            pltpu.SemaphoreType.DMA,
        ],
        compiler_params=pltpu.CompilerParams(needs_layout_passes=False),
    )
    return f(x, a, b, s)


def kernel(input_feat, x, conv_w, conv_b, bn_gamma, bn_beta,
           offset_w, offset_b, weight_w, weight_b):
    del input_feat
    cw = conv_w[:, 0, :].astype(jnp.float32)
    w8 = jnp.concatenate(
        [offset_w, weight_w, jnp.zeros((2, C), jnp.float32)], axis=0)
    b8 = jnp.concatenate(
        [offset_b, weight_b, jnp.zeros((2,), jnp.float32)])[:, None]
    a, b, s = _dense_stage(x, cw, conv_b[:, None], bn_gamma[:, None],
                           bn_beta[:, None], w8, b8)
    return _sc_stage(x, a, b, s)
